# Initial kernel scaffold; baseline (speedup 1.0000x reference)
import functools

import jax
import jax.numpy as jnp
from jax import lax
from jax.experimental import pallas as pl
from jax.experimental.pallas import tpu as pltpu
from jax.experimental.pallas import tpu_sc as plsc

NC = 2
NS = 16
LANES = 16
C = 80
ZR = 25


@functools.lru_cache(maxsize=None)
def _make_sc_agg(N, D, E):
    NW = NC * NS
    EPT = E // NW
    J = EPT // C
    ROWS_PT = N // NS
    assert EPT * NW == E and J * C == EPT and ROWS_PT * NS == N
    assert ROWS_PT % ZR == 0

    mesh = plsc.VectorSubcoreMesh(
        core_axis_name="c", subcore_axis_name="s",
        num_cores=NC, num_subcores=NS)

    @functools.partial(
        pl.kernel,
        out_type=(jax.ShapeDtypeStruct((NC, N, D), jnp.float32),
                  jax.ShapeDtypeStruct((NC, N, LANES), jnp.float32)),
        mesh=mesh,
        scratch_types=[
            pltpu.VMEM((J, C), jnp.int32),
            pltpu.VMEM((J, C), jnp.int32),
            pltpu.VMEM((C, D), jnp.float32),
            pltpu.VMEM((C, LANES), jnp.float32),
            pltpu.VMEM((ZR, D), jnp.float32),
            pltpu.VMEM((ZR, LANES), jnp.float32),
            pltpu.VMEM_SHARED((N, D), jnp.float32),
            pltpu.VMEM_SHARED((N, LANES), jnp.float32),
            pltpu.SemaphoreType.DMA,
        ])
    def sc_agg(h_hbm, src_hbm, dst_hbm, psum_hbm, pdeg_hbm,
               srcv, dstv, gbuf, ones, zbuf, dzbuf, acc, deg, sem):
        c = lax.axis_index("c")
        s = lax.axis_index("s")
        t = c * NS + s

        @pl.loop(0, ZR)
        def _(i):
            dzbuf[i, :] = jnp.zeros((LANES,), jnp.float32)

            @pl.loop(0, D, step=LANES)
            def _(j):
                zbuf[i, pl.ds(j, LANES)] = jnp.zeros((LANES,), jnp.float32)

        @pl.loop(0, C)
        def _(i):
            ones[i, :] = jnp.ones((LANES,), jnp.float32)

        base = s * ROWS_PT

        @pl.loop(0, ROWS_PT, step=ZR)
        def _(r):
            pltpu.sync_copy(zbuf, acc.at[pl.ds(base + r, ZR)])
            pltpu.sync_copy(dzbuf, deg.at[pl.ds(base + r, ZR)])

        pltpu.sync_copy(src_hbm.at[pl.ds(t * J, J)], srcv)
        pltpu.sync_copy(dst_hbm.at[pl.ds(t * J, J)], dstv)

        plsc.subcore_barrier()

        @pl.loop(0, J)
        def _(j):
            pltpu.async_copy(h_hbm.at[srcv.at[j]], gbuf, sem).wait()
            pltpu.sync_copy(gbuf, acc.at[dstv.at[j]], add=True)
            pltpu.sync_copy(ones, deg.at[dstv.at[j]], add=True)

        plsc.subcore_barrier()

        pltpu.sync_copy(acc.at[pl.ds(base, ROWS_PT)],
                        psum_hbm.at[c, pl.ds(base, ROWS_PT)])
        pltpu.sync_copy(deg.at[pl.ds(base, ROWS_PT)],
                        pdeg_hbm.at[c, pl.ds(base, ROWS_PT)])

    return sc_agg


@functools.lru_cache(maxsize=None)
def _make_tc_mlp(N, D, H, O):
    R = 2000
    assert N % R == 0

    def body(h_ref, ps_ref, pd_ref, w1_ref, b1_ref, w2_ref, b2_ref, out_ref):
        ssum = ps_ref[0] + ps_ref[1]
        dcol = pd_ref[0, :, 0] + pd_ref[1, :, 0]
        hn = ssum / jnp.maximum(dcol, 1.0)[:, None]
        z = (jnp.dot(h_ref[...], w1_ref[:D],
                     preferred_element_type=jnp.float32,
                     precision=lax.Precision.HIGHEST)
             + jnp.dot(hn, w1_ref[D:],
                       preferred_element_type=jnp.float32,
                       precision=lax.Precision.HIGHEST)
             + b1_ref[...])
        z2 = jnp.dot(z, w2_ref[...],
                     preferred_element_type=jnp.float32,
                     precision=lax.Precision.HIGHEST) + b2_ref[...]
        out_ref[...] = jnp.maximum(z2, 0.0)

    return pl.pallas_call(
        body,
        grid=(N // R,),
        in_specs=[
            pl.BlockSpec((R, D), lambda i: (i, 0)),
            pl.BlockSpec((NC, R, D), lambda i: (0, i, 0)),
            pl.BlockSpec((NC, R, LANES), lambda i: (0, i, 0)),
            pl.BlockSpec((2 * D, H), lambda i: (0, 0)),
            pl.BlockSpec((1, H), lambda i: (0, 0)),
            pl.BlockSpec((H, O), lambda i: (0, 0)),
            pl.BlockSpec((1, O), lambda i: (0, 0)),
        ],
        out_specs=pl.BlockSpec((R, O), lambda i: (i, 0)),
        out_shape=jax.ShapeDtypeStruct((N, O), jnp.float32),
    )


def kernel(h, edge_index, W1, b1, W2, b2):
    N, D = h.shape
    E = edge_index.shape[1]
    H = W1.shape[1]
    O = W2.shape[1]

    src = edge_index[0].astype(jnp.int32).reshape(E // C, C)
    dst = edge_index[1].astype(jnp.int32).reshape(E // C, C)
    h32 = h.astype(jnp.float32)

    psum, pdeg = _make_sc_agg(N, D, E)(h32, src, dst)
    out = _make_tc_mlp(N, D, H, O)(
        h32, psum, pdeg,
        W1.astype(jnp.float32), b1.astype(jnp.float32).reshape(1, H),
        W2.astype(jnp.float32), b2.astype(jnp.float32).reshape(1, O))
    return out


# speedup vs baseline: 12.2659x; 12.2659x over previous
import dataclasses
import functools

import numpy as np
import jax
import jax.numpy as jnp
from jax import lax
from jax.experimental import pallas as pl
from jax.experimental.pallas import tpu as pltpu
from jax.experimental.pallas import tpu_sc as plsc

NC = 2
NS = 16
LANES = 16
C = 80
ZR = 24
GB = 5


def _i32_loop(lo, hi, step, body):
    n = (hi - lo) // step

    def f(i, carry):
        body(jnp.int32(lo) + i * jnp.int32(step))
        return carry

    lax.fori_loop(jnp.int32(0), jnp.int32(n), f, jnp.int32(0))


@functools.lru_cache(maxsize=None)
def _make_sc_agg(N, D, E):
    NW = NC * NS
    EPT = E // NW
    J = EPT // C
    G = J // GB
    W = (N // NS) // 8 * 8
    RG = 8
    assert EPT * NW == E and G * GB == J and J * C == EPT
    assert (NS - 1) * W + RG * C == N

    mesh = plsc.VectorSubcoreMesh(
        core_axis_name="c", subcore_axis_name="s",
        num_cores=NC, num_subcores=NS)

    cp = pltpu.CompilerParams()
    if "needs_layout_passes" in pltpu.CompilerParams.__dataclass_fields__:
        cp = dataclasses.replace(cp, needs_layout_passes=False)

    @functools.partial(
        pl.kernel,
        out_type=(jax.ShapeDtypeStruct((NC, N, D), jnp.float32),
                  jax.ShapeDtypeStruct((NW, 1, N), jnp.float32)),
        mesh=mesh, compiler_params=cp,
        scratch_types=[
            pltpu.VMEM((GB, C), jnp.int32),
            pltpu.VMEM((GB, C), jnp.int32),
            pltpu.VMEM((C, D), jnp.float32),
            pltpu.VMEM((LANES, D), jnp.float32),
            pltpu.VMEM((C // LANES, LANES), jnp.int32),
            pltpu.VMEM((C,), jnp.int32),
            pltpu.VMEM((1, N), jnp.float32),
            pltpu.VMEM_SHARED((N, D), jnp.float32),
            pltpu.SemaphoreType.DMA,
        ])
    def sc_agg(h_hbm, src_hbm, dst_hbm, psum_hbm, pdeg_hbm,
               srcv, dstv, gbuf, zbuf16, ridx2, ridx, degloc, acc, sem):
        c = lax.axis_index("c").astype(jnp.int32)
        s = lax.axis_index("s").astype(jnp.int32)
        t = c * jnp.int32(NS) + s
        lane = lax.iota(jnp.int32, LANES)
        zero16f = jnp.zeros((LANES,), jnp.float32)
        zero16i = jnp.zeros((LANES,), jnp.int32)
        ones16 = jnp.ones((LANES,), jnp.float32)
        z32 = jnp.int32(0)

        def init_zrow(i):
            def inner(j):
                zbuf16[i, pl.ds(j, LANES)] = zero16f

            _i32_loop(0, D, LANES, inner)

        _i32_loop(0, LANES, 1, init_zrow)

        def init_deg(i):
            degloc[z32, pl.ds(i, LANES)] = zero16f

        _i32_loop(0, N, LANES, init_deg)

        base = s * jnp.int32(W)

        def zero_group(g):
            row0 = base + g * jnp.int32(C)
            for j in range(C // LANES):
                ridx2[jnp.int32(j), :] = lane + row0 + jnp.int32(j * LANES)
            for j in range(C // LANES):
                pltpu.sync_copy(zbuf16, acc.at[ridx2.at[jnp.int32(j)]])

        _i32_loop(0, RG, 1, zero_group)

        plsc.subcore_barrier()

        def group(g):
            tg = t * jnp.int32(G) + g
            pltpu.sync_copy(src_hbm.at[tg], srcv)
            pltpu.sync_copy(dst_hbm.at[tg], dstv)
            for k in range(GB):
                k32 = jnp.int32(k)
                pltpu.async_copy(h_hbm.at[srcv.at[k32]], gbuf, sem).wait()
                pltpu.sync_copy(gbuf, acc.at[dstv.at[k32]], add=True)
                for j in range(C // LANES):
                    idx16 = dstv[k32, pl.ds(jnp.int32(j * LANES), LANES)]
                    plsc.addupdate_scatter(degloc, [zero16i, idx16], ones16)

        _i32_loop(0, G, 1, group)

        plsc.subcore_barrier()

        def fill_ridx(row0):
            for j in range(C // LANES):
                ridx[pl.ds(jnp.int32(j * LANES), LANES)] = (
                    lane + row0 + jnp.int32(j * LANES))

        def write_group(g):
            row0 = base + g * jnp.int32(C)
            fill_ridx(row0)
            pltpu.sync_copy(acc.at[ridx], gbuf)
            pltpu.sync_copy(gbuf, psum_hbm.at[c, pl.ds(row0, C)])

        _i32_loop(0, RG, 1, write_group)

        pltpu.sync_copy(degloc, pdeg_hbm.at[t])

    return sc_agg


@functools.lru_cache(maxsize=None)
def _make_tc_mlp(N, D, H, O):
    _Z = np.int32(0)

    def body(h_ref, ps_ref, pd_ref, w1_ref, b1_ref, w2_ref, b2_ref, out_ref):
        ssum = ps_ref[0] + ps_ref[1]
        dcol = jnp.sum(pd_ref[...], axis=0)
        hn = ssum / jnp.maximum(dcol, 1.0)[:, None]
        z = (jnp.dot(h_ref[...], w1_ref[:D],
                     preferred_element_type=jnp.float32,
                     precision=lax.Precision.HIGHEST)
             + jnp.dot(hn, w1_ref[D:],
                       preferred_element_type=jnp.float32,
                       precision=lax.Precision.HIGHEST)
             + b1_ref[...])
        z2 = jnp.dot(z, w2_ref[...],
                     preferred_element_type=jnp.float32,
                     precision=lax.Precision.HIGHEST) + b2_ref[...]
        out_ref[...] = jnp.maximum(z2, 0.0)

    return pl.pallas_call(
        body,
        out_shape=jax.ShapeDtypeStruct((N, O), jnp.float32),
    )


def kernel(h, edge_index, W1, b1, W2, b2):
    N, D = h.shape
    E = edge_index.shape[1]
    H = W1.shape[1]
    O = W2.shape[1]

    NW = NC * NS
    J = E // (NW * C)
    src = edge_index[0].astype(jnp.int32).reshape(NW * (J // GB), GB, C)
    dst = edge_index[1].astype(jnp.int32).reshape(NW * (J // GB), GB, C)
    h32 = h.astype(jnp.float32)

    psum, pdeg = _make_sc_agg(N, D, E)(h32, src, dst)
    pdeg = pdeg.reshape(NC * NS, N)
    out = _make_tc_mlp(N, D, H, O)(
        h32, psum, pdeg,
        W1.astype(jnp.float32), b1.astype(jnp.float32).reshape(1, H),
        W2.astype(jnp.float32), b2.astype(jnp.float32).reshape(1, O))
    return out.astype(jnp.result_type(h, W1, W2))
